# Initial kernel scaffold; baseline (speedup 1.0000x reference)
import jax
import jax.numpy as jnp
from jax.experimental import pallas as pl


def kernel(X, edge_index, edge_attr, bn_in_g, bn_in_b, W_e0, b_e0, W_nn0, b_nn0, bn0_g, bn0_b, W_e1, b_e1, W_nn1, b_nn1, bn1_g, bn1_b, fc_W):
    raise NotImplementedError("write your pallas kernel here")


# speedup vs baseline: 1.8523x; 1.8523x over previous
import functools

import jax
import jax.numpy as jnp
from jax import lax
from jax.experimental import pallas as pl
from jax.experimental.pallas import tpu as pltpu
from jax.experimental.pallas import tpu_sc as plsc

N = 10000
E = 320000
D = 128

NC = 2
NS = 16
NW = NC * NS

CHUNK = 128
EPW = 10240
E_PAD = EPW * NW
N_CHUNKS = EPW // CHUNK
ACC_R = 10112
ROWS_PER_TILE = ACC_R // NS
DUMMY_ROW = N

BE = 2560
N_REAL_BLOCKS = E // BE
N_BLOCKS = E_PAD // BE


def _bn_body(x_ref, g_ref, b_ref, o_ref):
    x = x_ref[...]
    mu = jnp.mean(x, axis=0)
    var = jnp.mean((x - mu) ** 2, axis=0)
    scale = g_ref[...] * lax.rsqrt(var + 1e-5)
    o_ref[...] = x * scale + (b_ref[...] - mu * scale)


def _bn(x, g, b):
    return pl.pallas_call(
        _bn_body,
        out_shape=jax.ShapeDtypeStruct((N, D), jnp.float32),
    )(x, g, b)


def _proj_body(ea_ref, w0_ref, b0_ref, w1_ref, b1_ref, p0_ref, p1_ref):
    ea = ea_ref[...]
    p0_ref[...] = jnp.dot(ea, w0_ref[...], preferred_element_type=jnp.float32) + b0_ref[...]
    p1_ref[...] = jnp.dot(ea, w1_ref[...], preferred_element_type=jnp.float32) + b1_ref[...]


def _proj(edge_attr, w0t, b0, w1t, b1):
    return pl.pallas_call(
        _proj_body,
        grid=(N_BLOCKS,),
        in_specs=[
            pl.BlockSpec((BE, D), lambda i: (jnp.minimum(i, N_REAL_BLOCKS - 1), 0)),
            pl.BlockSpec((D, D), lambda i: (0, 0)),
            pl.BlockSpec((D,), lambda i: (0,)),
            pl.BlockSpec((D, D), lambda i: (0, 0)),
            pl.BlockSpec((D,), lambda i: (0,)),
        ],
        out_specs=[
            pl.BlockSpec((BE, D), lambda i: (i, 0)),
            pl.BlockSpec((BE, D), lambda i: (i, 0)),
        ],
        out_shape=[
            jax.ShapeDtypeStruct((E_PAD, D), jnp.float32),
            jax.ShapeDtypeStruct((E_PAD, D), jnp.float32),
        ],
    )(edge_attr, w0t, b0, w1t, b1)


def _node_body(x_ref, part_ref, wt_ref, b_ref, g_ref, bb_ref, o_ref):
    h = x_ref[...] + part_ref[0, :N, :] + part_ref[1, :N, :]
    y = jnp.tanh(jnp.dot(h, wt_ref[...], preferred_element_type=jnp.float32) + b_ref[...])
    mu = jnp.mean(y, axis=0)
    var = jnp.mean((y - mu) ** 2, axis=0)
    scale = g_ref[...] * lax.rsqrt(var + 1e-5)
    o_ref[...] = y * scale + (bb_ref[...] - mu * scale)


def _node(x, part, wt, b, g, bb):
    return pl.pallas_call(
        _node_body,
        out_shape=jax.ShapeDtypeStruct((N, D), jnp.float32),
    )(x, part, wt, b, g, bb)


def _node_final_body(x_ref, part_ref, wt_ref, b_ref, g_ref, bb_ref, fc_ref,
                     o2_ref, o3_ref):
    h = x_ref[...] + part_ref[0, :N, :] + part_ref[1, :N, :]
    y = jnp.tanh(jnp.dot(h, wt_ref[...], preferred_element_type=jnp.float32) + b_ref[...])
    mu = jnp.mean(y, axis=0)
    var = jnp.mean((y - mu) ** 2, axis=0)
    scale = g_ref[...] * lax.rsqrt(var + 1e-5)
    x2 = y * scale + (bb_ref[...] - mu * scale)
    o2_ref[...] = x2
    o3_ref[...] = jnp.tanh(jnp.dot(x2, fc_ref[...], preferred_element_type=jnp.float32))


def _node_final(x, part, wt, b, g, bb, fct):
    return pl.pallas_call(
        _node_final_body,
        out_shape=[
            jax.ShapeDtypeStruct((N, D), jnp.float32),
            jax.ShapeDtypeStruct((N, D), jnp.float32),
        ],
    )(x, part, wt, b, g, bb, fct)


def _edge_body(x_hbm, p_hbm, src_hbm, dst_hbm, zeros_hbm, out_hbm,
               srcbuf, dstbuf, pbuf, xgbuf, acc, sem):
    c = lax.axis_index("c")
    s = lax.axis_index("s")
    wid = s * NC + c

    r0 = s * ROWS_PER_TILE
    pltpu.sync_copy(zeros_hbm.at[pl.ds(r0, ROWS_PER_TILE)],
                    acc.at[pl.ds(r0, ROWS_PER_TILE)])
    plsc.subcore_barrier()

    base = wid * EPW
    irow_base = wid * (EPW // 128)

    def chunk_body(k, carry):
        off = base + k * CHUNK
        irow = irow_base + k * (CHUNK // 128)
        pltpu.sync_copy(src_hbm.at[pl.ds(irow, CHUNK // 128)], srcbuf)
        pltpu.sync_copy(dst_hbm.at[pl.ds(irow, CHUNK // 128)], dstbuf)
        pltpu.sync_copy(p_hbm.at[pl.ds(off, CHUNK)], pbuf)
        for g in range(CHUNK // 128):
            pltpu.async_copy(x_hbm.at[srcbuf.at[g]],
                             xgbuf.at[pl.ds(g * 128, 128)], sem).wait()

        def row_body(i, carry2):
            for j in range(D // 16):
                sl = pl.ds(j * 16, 16)
                xgbuf[i, sl] = jnp.maximum(xgbuf[i, sl] + pbuf[i, sl], 0.0)
            return carry2

        lax.fori_loop(0, CHUNK, row_body, 0, unroll=False)

        for g in range(CHUNK // 128):
            pltpu.sync_copy(xgbuf.at[pl.ds(g * 128, 128)],
                            acc.at[dstbuf.at[g]], add=True)
        return carry

    lax.fori_loop(0, N_CHUNKS, chunk_body, 0, unroll=False)

    plsc.subcore_barrier()
    pltpu.sync_copy(acc.at[pl.ds(r0, ROWS_PER_TILE)],
                    out_hbm.at[c, pl.ds(r0, ROWS_PER_TILE)])


@functools.partial(jax.jit, static_argnames=())
def _edge_sc(x, p, src2, dst2, zeros):
    mesh = plsc.VectorSubcoreMesh(core_axis_name="c", subcore_axis_name="s")
    return pl.kernel(
        _edge_body,
        out_type=jax.ShapeDtypeStruct((NC, ACC_R, D), jnp.float32),
        mesh=mesh,
        scratch_types=[
            pltpu.VMEM((CHUNK // 128, 128), jnp.int32),
            pltpu.VMEM((CHUNK // 128, 128), jnp.int32),
            pltpu.VMEM((CHUNK, D), jnp.float32),
            pltpu.VMEM((CHUNK, D), jnp.float32),
            pltpu.VMEM_SHARED((ACC_R, D), jnp.float32),
            pltpu.SemaphoreType.DMA,
        ],
    )(x, p, src2, dst2, zeros)


def kernel(X, edge_index, edge_attr, bn_in_g, bn_in_b, W_e0, b_e0, W_nn0,
           b_nn0, bn0_g, bn0_b, W_e1, b_e1, W_nn1, b_nn1, bn1_g, bn1_b, fc_W):
    src = edge_index[0]
    dst = edge_index[1]
    pad = E_PAD - E
    src2 = jnp.concatenate([src, jnp.zeros((pad,), jnp.int32)]).reshape(-1, 128)
    dst2 = jnp.concatenate(
        [dst, jnp.full((pad,), DUMMY_ROW, jnp.int32)]).reshape(-1, 128)
    zeros = jnp.zeros((ACC_R, D), jnp.float32)

    x0 = _bn(X, bn_in_g, bn_in_b)
    p0, p1 = _proj(edge_attr, W_e0.T, b_e0, W_e1.T, b_e1)

    part0 = _edge_sc(x0, p0, src2, dst2, zeros)
    x1 = _node(x0, part0, W_nn0.T, b_nn0, bn0_g, bn0_b)

    part1 = _edge_sc(x1, p1, src2, dst2, zeros)
    x2, x3 = _node_final(x1, part1, W_nn1.T, b_nn1, bn1_g, bn1_b, fc_W.T)

    return jnp.concatenate([x1, x2, x3], axis=-1)


# speedup vs baseline: 2.1334x; 1.1517x over previous
import functools

import jax
import jax.numpy as jnp
from jax import lax
from jax.experimental import pallas as pl
from jax.experimental.pallas import tpu as pltpu
from jax.experimental.pallas import tpu_sc as plsc

N = 10000
E = 320000
D = 128

NC = 2
NS = 16
NW = NC * NS

CHUNK = 128
EPW = 10240
E_PAD = EPW * NW
N_CHUNKS = EPW // CHUNK
ACC_R = 10112
ROWS_PER_TILE = ACC_R // NS
DUMMY_ROW = N

BE = 2560
N_REAL_BLOCKS = E // BE
N_BLOCKS = E_PAD // BE


def _bn_body(x_ref, g_ref, b_ref, o_ref):
    x = x_ref[...]
    mu = jnp.mean(x, axis=0)
    var = jnp.mean((x - mu) ** 2, axis=0)
    scale = g_ref[...] * lax.rsqrt(var + 1e-5)
    o_ref[...] = x * scale + (b_ref[...] - mu * scale)


def _bn(x, g, b):
    return pl.pallas_call(
        _bn_body,
        out_shape=jax.ShapeDtypeStruct((N, D), jnp.float32),
    )(x, g, b)


def _proj_body(ea_ref, w0_ref, b0_ref, w1_ref, b1_ref, p0_ref, p1_ref):
    ea = ea_ref[...]
    p0_ref[...] = jnp.dot(ea, w0_ref[...], preferred_element_type=jnp.float32) + b0_ref[...]
    p1_ref[...] = jnp.dot(ea, w1_ref[...], preferred_element_type=jnp.float32) + b1_ref[...]


def _proj(edge_attr, w0t, b0, w1t, b1):
    return pl.pallas_call(
        _proj_body,
        grid=(N_BLOCKS,),
        in_specs=[
            pl.BlockSpec((BE, D), lambda i: (jnp.minimum(i, N_REAL_BLOCKS - 1), 0)),
            pl.BlockSpec((D, D), lambda i: (0, 0)),
            pl.BlockSpec((D,), lambda i: (0,)),
            pl.BlockSpec((D, D), lambda i: (0, 0)),
            pl.BlockSpec((D,), lambda i: (0,)),
        ],
        out_specs=[
            pl.BlockSpec((BE, D), lambda i: (i, 0)),
            pl.BlockSpec((BE, D), lambda i: (i, 0)),
        ],
        out_shape=[
            jax.ShapeDtypeStruct((E_PAD, D), jnp.float32),
            jax.ShapeDtypeStruct((E_PAD, D), jnp.float32),
        ],
    )(edge_attr, w0t, b0, w1t, b1)


def _node_body(x_ref, part_ref, wt_ref, b_ref, g_ref, bb_ref, o_ref):
    h = x_ref[...] + part_ref[0, :N, :] + part_ref[1, :N, :]
    y = jnp.tanh(jnp.dot(h, wt_ref[...], preferred_element_type=jnp.float32) + b_ref[...])
    mu = jnp.mean(y, axis=0)
    var = jnp.mean((y - mu) ** 2, axis=0)
    scale = g_ref[...] * lax.rsqrt(var + 1e-5)
    o_ref[...] = y * scale + (bb_ref[...] - mu * scale)


def _node(x, part, wt, b, g, bb):
    return pl.pallas_call(
        _node_body,
        out_shape=jax.ShapeDtypeStruct((N, D), jnp.float32),
    )(x, part, wt, b, g, bb)


def _node_final_body(x_ref, part_ref, wt_ref, b_ref, g_ref, bb_ref, fc_ref,
                     o2_ref, o3_ref):
    h = x_ref[...] + part_ref[0, :N, :] + part_ref[1, :N, :]
    y = jnp.tanh(jnp.dot(h, wt_ref[...], preferred_element_type=jnp.float32) + b_ref[...])
    mu = jnp.mean(y, axis=0)
    var = jnp.mean((y - mu) ** 2, axis=0)
    scale = g_ref[...] * lax.rsqrt(var + 1e-5)
    x2 = y * scale + (bb_ref[...] - mu * scale)
    o2_ref[...] = x2
    o3_ref[...] = jnp.tanh(jnp.dot(x2, fc_ref[...], preferred_element_type=jnp.float32))


def _node_final(x, part, wt, b, g, bb, fct):
    return pl.pallas_call(
        _node_final_body,
        out_shape=[
            jax.ShapeDtypeStruct((N, D), jnp.float32),
            jax.ShapeDtypeStruct((N, D), jnp.float32),
        ],
    )(x, part, wt, b, g, bb, fct)


def _edge_body(x_hbm, p_hbm, src_hbm, dst_hbm, zeros_hbm, out_hbm,
               srcb0, srcb1, dstb0, dstb1, pb0, pb1, xgbuf, acc,
               insem0, insem1, gsem):
    c = lax.axis_index("c")
    s = lax.axis_index("s")
    wid = s * NC + c

    r0 = s * ROWS_PER_TILE
    pltpu.sync_copy(zeros_hbm.at[pl.ds(r0, ROWS_PER_TILE)],
                    acc.at[pl.ds(r0, ROWS_PER_TILE)])

    srcb = [srcb0, srcb1]
    dstb = [dstb0, dstb1]
    pb = [pb0, pb1]
    insem = [insem0, insem1]

    base = wid * EPW
    irow0 = wid * N_CHUNKS

    def issue_in(k, b):
        pltpu.async_copy(src_hbm.at[pl.ds(irow0 + k, 1)], srcb[b], insem[b])
        pltpu.async_copy(dst_hbm.at[pl.ds(irow0 + k, 1)], dstb[b], insem[b])
        pltpu.async_copy(p_hbm.at[pl.ds(base + k * CHUNK, CHUNK)], pb[b], insem[b])

    def wait_in(b):
        pltpu.make_async_copy(src_hbm.at[pl.ds(irow0, 1)], srcb[b], insem[b]).wait()
        pltpu.make_async_copy(dst_hbm.at[pl.ds(irow0, 1)], dstb[b], insem[b]).wait()
        pltpu.make_async_copy(p_hbm.at[pl.ds(base, CHUNK)], pb[b], insem[b]).wait()

    issue_in(0, 0)
    plsc.subcore_barrier()

    def outer(kk, carry):
        for b in range(2):
            k = 2 * kk + b

            @pl.when(k + 1 < N_CHUNKS)
            def _():
                issue_in(k + 1, 1 - b)

            wait_in(b)
            pltpu.async_copy(x_hbm.at[srcb[b].at[0]], xgbuf, gsem).wait()

            def row_body(i, carry2):
                for j in range(D // 16):
                    sl = pl.ds(j * 16, 16)
                    xgbuf[i, sl] = jnp.maximum(xgbuf[i, sl] + pb[b][i, sl], 0.0)
                return carry2

            lax.fori_loop(0, CHUNK, row_body, 0, unroll=False)

            pltpu.sync_copy(xgbuf, acc.at[dstb[b].at[0]], add=True)
        return carry

    lax.fori_loop(0, N_CHUNKS // 2, outer, 0, unroll=False)

    plsc.subcore_barrier()
    pltpu.sync_copy(acc.at[pl.ds(r0, ROWS_PER_TILE)],
                    out_hbm.at[c, pl.ds(r0, ROWS_PER_TILE)])


@functools.partial(jax.jit, static_argnames=())
def _edge_sc(x, p, src2, dst2, zeros):
    mesh = plsc.VectorSubcoreMesh(core_axis_name="c", subcore_axis_name="s")
    return pl.kernel(
        _edge_body,
        out_type=jax.ShapeDtypeStruct((NC, ACC_R, D), jnp.float32),
        mesh=mesh,
        scratch_types=[
            pltpu.VMEM((1, 128), jnp.int32),
            pltpu.VMEM((1, 128), jnp.int32),
            pltpu.VMEM((1, 128), jnp.int32),
            pltpu.VMEM((1, 128), jnp.int32),
            pltpu.VMEM((CHUNK, D), jnp.float32),
            pltpu.VMEM((CHUNK, D), jnp.float32),
            pltpu.VMEM((CHUNK, D), jnp.float32),
            pltpu.VMEM_SHARED((ACC_R, D), jnp.float32),
            pltpu.SemaphoreType.DMA,
            pltpu.SemaphoreType.DMA,
            pltpu.SemaphoreType.DMA,
        ],
    )(x, p, src2, dst2, zeros)


def kernel(X, edge_index, edge_attr, bn_in_g, bn_in_b, W_e0, b_e0, W_nn0,
           b_nn0, bn0_g, bn0_b, W_e1, b_e1, W_nn1, b_nn1, bn1_g, bn1_b, fc_W):
    src = edge_index[0]
    dst = edge_index[1]
    pad = E_PAD - E
    src2 = jnp.concatenate([src, jnp.zeros((pad,), jnp.int32)]).reshape(-1, 128)
    pad_dst = DUMMY_ROW + jnp.arange(pad, dtype=jnp.int32) % (ACC_R - N)
    dst2 = jnp.concatenate([dst, pad_dst]).reshape(-1, 128)
    zeros = jnp.zeros((ACC_R, D), jnp.float32)

    x0 = _bn(X, bn_in_g, bn_in_b)
    p0, p1 = _proj(edge_attr, W_e0.T, b_e0, W_e1.T, b_e1)

    part0 = _edge_sc(x0, p0, src2, dst2, zeros)
    x1 = _node(x0, part0, W_nn0.T, b_nn0, bn0_g, bn0_b)

    part1 = _edge_sc(x1, p1, src2, dst2, zeros)
    x2, x3 = _node_final(x1, part1, W_nn1.T, b_nn1, bn1_g, bn1_b, fc_W.T)

    return jnp.concatenate([x1, x2, x3], axis=-1)


# speedup vs baseline: 2.7074x; 1.2691x over previous
import functools

import jax
import jax.numpy as jnp
from jax import lax
from jax.experimental import pallas as pl
from jax.experimental.pallas import tpu as pltpu
from jax.experimental.pallas import tpu_sc as plsc

N = 10000
E = 320000
D = 128

NC = 2
NS = 16
NW = NC * NS

CHUNK = 128
E_PAD = 327680
A0 = 120
A1 = (E_PAD // CHUNK - A0 * NS) // NS
EPW0 = A0 * CHUNK
EPW1 = A1 * CHUNK
ACC_R = 10112
ROWS_PER_TILE = ACC_R // NS
DUMMY_ROW = N

BE = 2560
N_REAL_BLOCKS = E // BE
N_BLOCKS = E_PAD // BE


def _bn_body(x_ref, g_ref, b_ref, o_ref):
    x = x_ref[...]
    mu = jnp.mean(x, axis=0)
    var = jnp.mean((x - mu) ** 2, axis=0)
    scale = g_ref[...] * lax.rsqrt(var + 1e-5)
    o_ref[...] = x * scale + (b_ref[...] - mu * scale)


def _bn(x, g, b):
    return pl.pallas_call(
        _bn_body,
        out_shape=jax.ShapeDtypeStruct((N, D), jnp.float32),
    )(x, g, b)


def _proj_body(ea_ref, w0_ref, b0_ref, w1_ref, b1_ref, p0_ref, p1_ref):
    ea = ea_ref[...]
    p0_ref[...] = jnp.dot(ea, w0_ref[...], preferred_element_type=jnp.float32) + b0_ref[...]
    p1_ref[...] = jnp.dot(ea, w1_ref[...], preferred_element_type=jnp.float32) + b1_ref[...]


def _proj(edge_attr, w0t, b0, w1t, b1):
    return pl.pallas_call(
        _proj_body,
        grid=(N_BLOCKS,),
        in_specs=[
            pl.BlockSpec((BE, D), lambda i: (jnp.minimum(i, N_REAL_BLOCKS - 1), 0)),
            pl.BlockSpec((D, D), lambda i: (0, 0)),
            pl.BlockSpec((D,), lambda i: (0,)),
            pl.BlockSpec((D, D), lambda i: (0, 0)),
            pl.BlockSpec((D,), lambda i: (0,)),
        ],
        out_specs=[
            pl.BlockSpec((BE, D), lambda i: (i, 0)),
            pl.BlockSpec((BE, D), lambda i: (i, 0)),
        ],
        out_shape=[
            jax.ShapeDtypeStruct((E_PAD, D), jnp.float32),
            jax.ShapeDtypeStruct((E_PAD, D), jnp.float32),
        ],
    )(edge_attr, w0t, b0, w1t, b1)


def _node_body(x_ref, part_ref, wt_ref, b_ref, g_ref, bb_ref, o_ref):
    h = x_ref[...] + part_ref[0, :N, :] + part_ref[1, :N, :]
    y = jnp.tanh(jnp.dot(h, wt_ref[...], preferred_element_type=jnp.float32) + b_ref[...])
    mu = jnp.mean(y, axis=0)
    var = jnp.mean((y - mu) ** 2, axis=0)
    scale = g_ref[...] * lax.rsqrt(var + 1e-5)
    o_ref[...] = y * scale + (bb_ref[...] - mu * scale)


def _node(x, part, wt, b, g, bb):
    return pl.pallas_call(
        _node_body,
        out_shape=jax.ShapeDtypeStruct((N, D), jnp.float32),
    )(x, part, wt, b, g, bb)


def _node_final_body(x_ref, part_ref, wt_ref, b_ref, g_ref, bb_ref, fc_ref,
                     o2_ref, o3_ref):
    h = x_ref[...] + part_ref[0, :N, :] + part_ref[1, :N, :]
    y = jnp.tanh(jnp.dot(h, wt_ref[...], preferred_element_type=jnp.float32) + b_ref[...])
    mu = jnp.mean(y, axis=0)
    var = jnp.mean((y - mu) ** 2, axis=0)
    scale = g_ref[...] * lax.rsqrt(var + 1e-5)
    x2 = y * scale + (bb_ref[...] - mu * scale)
    o2_ref[...] = x2
    o3_ref[...] = jnp.tanh(jnp.dot(x2, fc_ref[...], preferred_element_type=jnp.float32))


def _node_final(x, part, wt, b, g, bb, fct):
    return pl.pallas_call(
        _node_final_body,
        out_shape=[
            jax.ShapeDtypeStruct((N, D), jnp.float32),
            jax.ShapeDtypeStruct((N, D), jnp.float32),
        ],
    )(x, part, wt, b, g, bb, fct)


def _edge_body(x_hbm, p_hbm, src_hbm, dst_hbm, zeros_hbm, out_hbm,
               srcb, dstb, pbuf, xg0, xg1, acc,
               isem0, isem1, psem, gsem0, gsem1, ssem0, ssem1):
    c = lax.axis_index("c")
    s = lax.axis_index("s")

    r0 = s * ROWS_PER_TILE
    pltpu.sync_copy(zeros_hbm.at[pl.ds(r0, ROWS_PER_TILE)],
                    acc.at[pl.ds(r0, ROWS_PER_TILE)])

    xg = [xg0, xg1]
    isem = [isem0, isem1]
    gsem = [gsem0, gsem1]
    ssem = [ssem0, ssem1]

    nchunks = jnp.where(c == 0, A0, A1)
    base = pl.multiple_of(
        jnp.where(c == 0, s * EPW0, NS * EPW0 + s * EPW1), CHUNK)
    irow0 = jnp.where(c == 0, s * A0, NS * A0 + s * A1)

    def issue_idx(k, b, q):
        pltpu.async_copy(src_hbm.at[pl.ds(irow0 + k, 1)],
                         srcb.at[q], isem[b])
        pltpu.async_copy(dst_hbm.at[pl.ds(irow0 + k, 1)],
                         dstb.at[q], isem[b])

    def wait_idx(b):
        pltpu.make_async_copy(src_hbm.at[pl.ds(irow0, 1)],
                              srcb.at[0], isem[b]).wait()
        pltpu.make_async_copy(dst_hbm.at[pl.ds(irow0, 1)],
                              dstb.at[0], isem[b]).wait()

    def issue_p(k):
        pltpu.async_copy(p_hbm.at[pl.ds(base + k * CHUNK, CHUNK)], pbuf, psem)

    def wait_p():
        pltpu.make_async_copy(p_hbm.at[pl.ds(base, CHUNK)], pbuf, psem).wait()

    def issue_gather(k, b, q):
        pltpu.async_copy(x_hbm.at[srcb.at[q, 0]], xg[b], gsem[b])

    def wait_gather(b):
        pltpu.make_async_copy(x_hbm.at[srcb.at[0, 0]], xg[b], gsem[b]).wait()

    def issue_scatter(b, q):
        pltpu.async_copy(xg[b], acc.at[dstb.at[q, 0]], ssem[b], add=True)

    def wait_scatter(b):
        pltpu.make_async_copy(xg[b], acc.at[dstb.at[0, 0]], ssem[b]).wait()

    issue_idx(0, 0, 0)
    issue_idx(1, 1, 1)
    issue_p(0)
    plsc.subcore_barrier()
    wait_idx(0)
    issue_gather(0, 0, 0)

    def outer(kk, carry):
        for b in range(2):
            k = 2 * kk + b
            q = lax.rem(k, 3)
            qn = lax.rem(k + 1, 3)
            qf = lax.rem(k + 2, 3)

            wait_gather(b)
            wait_p()

            def row_body(i, carry2):
                for j in range(D // 16):
                    sl = pl.ds(j * 16, 16)
                    xg[b][i, sl] = jnp.maximum(xg[b][i, sl] + pbuf[i, sl], 0.0)
                return carry2

            lax.fori_loop(0, CHUNK, row_body, 0, unroll=False)

            issue_scatter(b, q)

            @pl.when(k + 1 < nchunks)
            def _():
                issue_p(k + 1)
                wait_idx(1 - b)

                @pl.when(k >= 1)
                def _():
                    wait_scatter(1 - b)
                issue_gather(k + 1, 1 - b, qn)

            @pl.when(k + 2 < nchunks)
            def _():
                issue_idx(k + 2, b, qf)
        return carry

    lax.fori_loop(0, nchunks // 2, outer, 0, unroll=False)

    wait_scatter(0)
    wait_scatter(1)
    plsc.subcore_barrier()
    pltpu.sync_copy(acc.at[pl.ds(r0, ROWS_PER_TILE)],
                    out_hbm.at[c, pl.ds(r0, ROWS_PER_TILE)])


@functools.partial(jax.jit, static_argnames=())
def _edge_sc(x, p, src2, dst2, zeros):
    mesh = plsc.VectorSubcoreMesh(core_axis_name="c", subcore_axis_name="s")
    return pl.kernel(
        _edge_body,
        out_type=jax.ShapeDtypeStruct((NC, ACC_R, D), jnp.float32),
        mesh=mesh,
        scratch_types=[
            pltpu.VMEM((3, 1, 128), jnp.int32),
            pltpu.VMEM((3, 1, 128), jnp.int32),
            pltpu.VMEM((CHUNK, D), jnp.float32),
            pltpu.VMEM((CHUNK, D), jnp.float32),
            pltpu.VMEM((CHUNK, D), jnp.float32),
            pltpu.VMEM_SHARED((ACC_R, D), jnp.float32),
            pltpu.SemaphoreType.DMA,
            pltpu.SemaphoreType.DMA,
            pltpu.SemaphoreType.DMA,
            pltpu.SemaphoreType.DMA,
            pltpu.SemaphoreType.DMA,
            pltpu.SemaphoreType.DMA,
            pltpu.SemaphoreType.DMA,
        ],
    )(x, p, src2, dst2, zeros)


def kernel(X, edge_index, edge_attr, bn_in_g, bn_in_b, W_e0, b_e0, W_nn0,
           b_nn0, bn0_g, bn0_b, W_e1, b_e1, W_nn1, b_nn1, bn1_g, bn1_b, fc_W):
    src = edge_index[0]
    dst = edge_index[1]
    pad = E_PAD - E
    pad_dst = DUMMY_ROW + jnp.arange(pad, dtype=jnp.int32) % (ACC_R - N)
    src2 = jnp.concatenate([src, jnp.zeros((pad,), jnp.int32)]).reshape(-1, 128)
    dst2 = jnp.concatenate([dst, pad_dst]).reshape(-1, 128)
    zeros = jnp.zeros((ACC_R, D), jnp.float32)

    x0 = _bn(X, bn_in_g, bn_in_b)
    p0, p1 = _proj(edge_attr, W_e0.T, b_e0, W_e1.T, b_e1)

    part0 = _edge_sc(x0, p0, src2, dst2, zeros)
    x1 = _node(x0, part0, W_nn0.T, b_nn0, bn0_g, bn0_b)

    part1 = _edge_sc(x1, p1, src2, dst2, zeros)
    x2, x3 = _node_final(x1, part1, W_nn1.T, b_nn1, bn1_g, bn1_b, fc_W.T)

    return jnp.concatenate([x1, x2, x3], axis=-1)
